# baseline (speedup 1.0000x reference)
import jax
import jax.numpy as jnp
from jax import lax
from jax.experimental import pallas as pl
from jax.experimental.pallas import tpu as pltpu
from jax.experimental.pallas import tpu_sc as plsc

_N = 10000
_E = 320000
_D = 128
_DH = _D // 2
_NC = 2
_NS = 16
_B = 128
_CH = 160
_EPT = _CH * _B
_EPAD = _NS * _EPT
_PADROWS = 240
_NP = _N + _PADROWS
_RPT = _NP // _NS
_NBUF = 5


def _fill_zero(buf):
    v = jnp.zeros((16,), dtype=jnp.float32)
    npack = _DH // 16

    def body(i, carry):
        buf[i // npack, pl.ds((i % npack) * 16, 16)] = v
        return carry

    lax.fori_loop(0, 128 * npack, body, 0)


def _seg_sum_sc(src_t2, dst_t, h_flat):
    mesh = plsc.VectorSubcoreMesh(core_axis_name="c", subcore_axis_name="s")

    def body(src_hbm, dst_hbm, h_hbm, out_hbm, sidx, didx, rows, agg_sh, gsem):
        c = lax.axis_index("c")
        s = lax.axis_index("s")
        pltpu.sync_copy(src_hbm.at[c * _NS + s], sidx)
        pltpu.sync_copy(dst_hbm.at[s], didx)
        _fill_zero(rows.at[0])
        r0 = s * _RPT
        for j in range(_RPT // 128):
            pltpu.sync_copy(rows.at[0], agg_sh.at[pl.ds(r0 + j * 128, 128)])
        plsc.subcore_barrier()
        for bb in range(_NBUF):
            pltpu.async_copy(h_hbm.at[sidx.at[bb]], rows.at[bb], gsem.at[bb])

        def outer(t, carry):
            for bb in range(_NBUF):
                g = t * _NBUF + bb
                pltpu.make_async_copy(
                    h_hbm.at[sidx.at[bb]], rows.at[bb], gsem.at[bb]
                ).wait()
                pltpu.sync_copy(rows.at[bb], agg_sh.at[didx.at[g]], add=True)

                @pl.when(t < _CH // _NBUF - 1)
                def _():
                    pltpu.async_copy(
                        h_hbm.at[sidx.at[g + _NBUF]], rows.at[bb], gsem.at[bb]
                    )
            return carry

        lax.fori_loop(0, _CH // _NBUF, outer, 0)
        plsc.subcore_barrier()
        pltpu.sync_copy(
            agg_sh.at[pl.ds(r0, _RPT)],
            out_hbm.at[pl.ds(r0, _RPT), pl.ds(c * _DH, _DH)],
        )

    f = pl.kernel(
        body,
        out_type=jax.ShapeDtypeStruct((_NP, _D), jnp.float32),
        mesh=mesh,
        compiler_params=pltpu.CompilerParams(use_tc_tiling_on_sc=False),
        scratch_types=[
            pltpu.VMEM((_CH, _B), jnp.int32),
            pltpu.VMEM((_CH, _B), jnp.int32),
            pltpu.VMEM((_NBUF, _B, _DH), jnp.float32),
            pltpu.VMEM_SHARED((_NP, _DH), jnp.float32),
            pltpu.SemaphoreType.DMA((_NBUF,)),
        ],
    )
    return f(src_t2, dst_t, h_flat)


def _inv_deg_tc(deg):
    blk = 2048

    def body(d_ref, o_ref):
        o_ref[...] = 1.0 / jnp.maximum(d_ref[...], 1.0)

    return pl.pallas_call(
        body,
        grid=(_NP // blk,),
        in_specs=[pl.BlockSpec((blk, _D), lambda i: (i, 0))],
        out_specs=pl.BlockSpec((blk, _D), lambda i: (i, 0)),
        out_shape=jax.ShapeDtypeStruct((_NP, _D), jnp.float32),
    )(deg)


def _layer_tc(h, agg, invd, w_s, w_n, bias, relu):
    blk = 2000

    def body(h_ref, a_ref, i_ref, ws_ref, wn_ref, b_ref, o_ref):
        mean = a_ref[...] * i_ref[...]
        acc = jnp.dot(h_ref[...], ws_ref[...], preferred_element_type=jnp.float32)
        acc = acc + jnp.dot(mean, wn_ref[...], preferred_element_type=jnp.float32)
        acc = acc + b_ref[...]
        if relu:
            acc = jnp.maximum(acc, 0.0)
        o_ref[...] = acc

    return pl.pallas_call(
        body,
        grid=(_N // blk,),
        in_specs=[
            pl.BlockSpec((blk, _D), lambda i: (i, 0)),
            pl.BlockSpec((blk, _D), lambda i: (i, 0)),
            pl.BlockSpec((blk, _D), lambda i: (i, 0)),
            pl.BlockSpec((_D, _D), lambda i: (0, 0)),
            pl.BlockSpec((_D, _D), lambda i: (0, 0)),
            pl.BlockSpec((1, _D), lambda i: (0, 0)),
        ],
        out_specs=pl.BlockSpec((blk, _D), lambda i: (i, 0)),
        out_shape=jax.ShapeDtypeStruct((_N, _D), jnp.float32),
    )(h, agg, invd, w_s, w_n, bias)


def kernel(x, edge_index, W_self, W_neigh, b, W_self_out, W_neigh_out, b_out):
    src = edge_index[0]
    dst = edge_index[1]
    pad_n = _EPAD - _E
    ar = jnp.arange(pad_n, dtype=jnp.int32)
    pad_src = (ar * 131) % _N
    pad_dst = _N + (ar % _PADROWS)
    src_t = jnp.concatenate([src, pad_src]).reshape(_NS, _CH, _B)
    dst_t = jnp.concatenate([dst, pad_dst]).reshape(_NS, _CH, _B)
    src_t2 = jnp.concatenate([2 * src_t, 2 * src_t + 1], axis=0)

    def seg_sum(h):
        return _seg_sum_sc(src_t2, dst_t, h.reshape(2 * _N, _DH))

    deg = seg_sum(jnp.ones((_N, _D), jnp.float32))
    invd = _inv_deg_tc(deg)

    w_s_out = jnp.zeros((_D, _D), jnp.float32).at[:, : W_self_out.shape[1]].set(W_self_out)
    w_n_out = jnp.zeros((_D, _D), jnp.float32).at[:, : W_neigh_out.shape[1]].set(W_neigh_out)
    b_o = jnp.zeros((1, _D), jnp.float32).at[0, : b_out.shape[0]].set(b_out)

    h = x
    n_hidden = W_self.shape[0]
    for i in range(n_hidden):
        agg = seg_sum(h)
        h = _layer_tc(h, agg, invd, W_self[i], W_neigh[i], b[i].reshape(1, _D),
                      relu=(i >= 1))
    agg = seg_sum(h)
    out = _layer_tc(h, agg, invd, w_s_out, w_n_out, b_o, relu=False)
    return out[:, : b_out.shape[0]]


# speedup vs baseline: 1.0335x; 1.0335x over previous
import jax
import jax.numpy as jnp
from jax import lax
from jax.experimental import pallas as pl
from jax.experimental.pallas import tpu as pltpu
from jax.experimental.pallas import tpu_sc as plsc

_N = 10000
_E = 320000
_D = 128
_DH = _D // 2
_NC = 2
_NS = 16
_B = 128
_CH = 160
_EPT = _CH * _B
_EPAD = _NS * _EPT
_PADROWS = 240
_NP = _N + _PADROWS
_RPT = _NP // _NS
_NBUF = 5


def _fill_zero(buf):
    v = jnp.zeros((16,), dtype=jnp.float32)
    npack = _DH // 16

    def body(i, carry):
        buf[i // npack, pl.ds((i % npack) * 16, 16)] = v
        return carry

    lax.fori_loop(0, 128 * npack, body, 0)


def _seg_sum_sc(src_t2, dst_t, h_flat):
    mesh = plsc.VectorSubcoreMesh(core_axis_name="c", subcore_axis_name="s")

    def body(src_hbm, dst_hbm, h_hbm, out_hbm, sidx, didx, rows, agg_sh, gsem):
        c = lax.axis_index("c")
        s = lax.axis_index("s")
        sc = pltpu.async_copy(src_hbm.at[c * _NS + s], sidx, gsem.at[0])
        dc = pltpu.async_copy(dst_hbm.at[s], didx, gsem.at[1])
        _fill_zero(rows.at[0])
        r0 = s * _RPT
        for j in range(_RPT // 128):
            pltpu.sync_copy(rows.at[0], agg_sh.at[pl.ds(r0 + j * 128, 128)])
        sc.wait()
        dc.wait()
        plsc.subcore_barrier()
        for bb in range(_NBUF):
            pltpu.async_copy(h_hbm.at[sidx.at[bb]], rows.at[bb], gsem.at[bb])

        def outer(t, carry):
            for bb in range(_NBUF):
                g = t * _NBUF + bb
                pltpu.make_async_copy(
                    h_hbm.at[sidx.at[bb]], rows.at[bb], gsem.at[bb]
                ).wait()
                pltpu.sync_copy(rows.at[bb], agg_sh.at[didx.at[g]], add=True)

                @pl.when(t < _CH // _NBUF - 1)
                def _():
                    pltpu.async_copy(
                        h_hbm.at[sidx.at[g + _NBUF]], rows.at[bb], gsem.at[bb]
                    )
            return carry

        lax.fori_loop(0, _CH // _NBUF, outer, 0)
        plsc.subcore_barrier()
        pltpu.sync_copy(
            agg_sh.at[pl.ds(r0, _RPT)],
            out_hbm.at[pl.ds(r0, _RPT), pl.ds(c * _DH, _DH)],
        )

    f = pl.kernel(
        body,
        out_type=jax.ShapeDtypeStruct((_NP, _D), jnp.float32),
        mesh=mesh,
        compiler_params=pltpu.CompilerParams(use_tc_tiling_on_sc=False),
        scratch_types=[
            pltpu.VMEM((_CH, _B), jnp.int32),
            pltpu.VMEM((_CH, _B), jnp.int32),
            pltpu.VMEM((_NBUF, _B, _DH), jnp.float32),
            pltpu.VMEM_SHARED((_NP, _DH), jnp.float32),
            pltpu.SemaphoreType.DMA((_NBUF,)),
        ],
    )
    return f(src_t2, dst_t, h_flat)


def _inv_deg_tc(deg):
    blk = 2048

    def body(d_ref, o_ref):
        o_ref[...] = 1.0 / jnp.maximum(d_ref[...], 1.0)

    return pl.pallas_call(
        body,
        grid=(_NP // blk,),
        in_specs=[pl.BlockSpec((blk, _D), lambda i: (i, 0))],
        out_specs=pl.BlockSpec((blk, _D), lambda i: (i, 0)),
        out_shape=jax.ShapeDtypeStruct((_NP, _D), jnp.float32),
    )(deg)


def _layer_tc(h, agg, invd, w_s, w_n, bias, relu, out_w=_D):
    blk = 2000

    def body(h_ref, a_ref, i_ref, ws_ref, wn_ref, b_ref, o_ref):
        mean = a_ref[...] * i_ref[...]
        acc = jnp.dot(h_ref[...], ws_ref[...], preferred_element_type=jnp.float32)
        acc = acc + jnp.dot(mean, wn_ref[...], preferred_element_type=jnp.float32)
        acc = acc + b_ref[...]
        if relu:
            acc = jnp.maximum(acc, 0.0)
        o_ref[...] = acc[:, :out_w]

    return pl.pallas_call(
        body,
        grid=(_N // blk,),
        in_specs=[
            pl.BlockSpec((blk, _D), lambda i: (i, 0)),
            pl.BlockSpec((blk, _D), lambda i: (i, 0)),
            pl.BlockSpec((blk, _D), lambda i: (i, 0)),
            pl.BlockSpec((_D, _D), lambda i: (0, 0)),
            pl.BlockSpec((_D, _D), lambda i: (0, 0)),
            pl.BlockSpec((1, _D), lambda i: (0, 0)),
        ],
        out_specs=pl.BlockSpec((blk, out_w), lambda i: (i, 0)),
        out_shape=jax.ShapeDtypeStruct((_N, out_w), jnp.float32),
    )(h, agg, invd, w_s, w_n, bias)


def kernel(x, edge_index, W_self, W_neigh, b, W_self_out, W_neigh_out, b_out):
    src = edge_index[0]
    dst = edge_index[1]
    pad_n = _EPAD - _E
    ar = jnp.arange(pad_n, dtype=jnp.int32)
    pad_src = (ar * 131) % _N
    pad_dst = _N + (ar % _PADROWS)
    src_t = jnp.concatenate([src, pad_src]).reshape(_NS, _CH, _B)
    dst_t = jnp.concatenate([dst, pad_dst]).reshape(_NS, _CH, _B)
    src_t2 = jnp.concatenate([2 * src_t, 2 * src_t + 1], axis=0)

    def seg_sum(h):
        return _seg_sum_sc(src_t2, dst_t, h.reshape(2 * _N, _DH))

    deg = seg_sum(jnp.ones((_N, _D), jnp.float32))
    invd = _inv_deg_tc(deg)

    w_s_out = jnp.zeros((_D, _D), jnp.float32).at[:, : W_self_out.shape[1]].set(W_self_out)
    w_n_out = jnp.zeros((_D, _D), jnp.float32).at[:, : W_neigh_out.shape[1]].set(W_neigh_out)
    b_o = jnp.zeros((1, _D), jnp.float32).at[0, : b_out.shape[0]].set(b_out)

    h = x
    n_hidden = W_self.shape[0]
    for i in range(n_hidden):
        agg = seg_sum(h)
        h = _layer_tc(h, agg, invd, W_self[i], W_neigh[i], b[i].reshape(1, _D),
                      relu=(i >= 1))
    agg = seg_sum(h)
    return _layer_tc(h, agg, invd, w_s_out, w_n_out, b_o, relu=False,
                     out_w=b_out.shape[0])


# speedup vs baseline: 1.0366x; 1.0030x over previous
import jax
import jax.numpy as jnp
from jax import lax
from jax.experimental import pallas as pl
from jax.experimental.pallas import tpu as pltpu
from jax.experimental.pallas import tpu_sc as plsc

_N = 10000
_E = 320000
_D = 128
_DH = _D // 2
_NC = 2
_NS = 16
_B = 128
_CH = 160
_EPT = _CH * _B
_EPAD = _NS * _EPT
_PADROWS = 240
_NP = _N + _PADROWS
_RPT = _NP // _NS
_NBUF = 5


def _fill_zero(buf):
    v = jnp.zeros((16,), dtype=jnp.float32)
    npack = _DH // 16

    def body(i, carry):
        buf[i // npack, pl.ds((i % npack) * 16, 16)] = v
        return carry

    lax.fori_loop(0, 128 * npack, body, 0)


def _seg_sum_sc(src_t2, dst_t, h_flat):
    mesh = plsc.VectorSubcoreMesh(core_axis_name="c", subcore_axis_name="s")

    def body(src_hbm, dst_hbm, h_hbm, out_hbm, sidx, didx, rows, agg_sh, gsem):
        c = lax.axis_index("c")
        s = lax.axis_index("s")
        sc = pltpu.async_copy(src_hbm.at[c * _NS + s], sidx, gsem.at[0])
        dc = pltpu.async_copy(dst_hbm.at[s], didx, gsem.at[1])
        _fill_zero(rows.at[0])
        r0 = s * _RPT
        for j in range(_RPT // 128):
            pltpu.sync_copy(rows.at[0], agg_sh.at[pl.ds(r0 + j * 128, 128)])
        sc.wait()
        dc.wait()
        plsc.subcore_barrier()
        for bb in range(_NBUF):
            pltpu.async_copy(h_hbm.at[sidx.at[bb]], rows.at[bb], gsem.at[bb])

        def outer(t, carry):
            for bb in range(_NBUF):
                g = t * _NBUF + bb
                pltpu.make_async_copy(
                    h_hbm.at[sidx.at[bb]], rows.at[bb], gsem.at[bb]
                ).wait()
                pltpu.sync_copy(rows.at[bb], agg_sh.at[didx.at[g]], add=True)

                @pl.when(t < _CH // _NBUF - 1)
                def _():
                    pltpu.async_copy(
                        h_hbm.at[sidx.at[g + _NBUF]], rows.at[bb], gsem.at[bb]
                    )
            return carry

        lax.fori_loop(0, _CH // _NBUF, outer, 0)
        plsc.subcore_barrier()
        pltpu.sync_copy(
            agg_sh.at[pl.ds(r0, _RPT)],
            out_hbm.at[pl.ds(r0, _RPT), pl.ds(c * _DH, _DH)],
        )

    f = pl.kernel(
        body,
        out_type=jax.ShapeDtypeStruct((_NP, _D), jnp.float32),
        mesh=mesh,
        compiler_params=pltpu.CompilerParams(use_tc_tiling_on_sc=False),
        scratch_types=[
            pltpu.VMEM((_CH, _B), jnp.int32),
            pltpu.VMEM((_CH, _B), jnp.int32),
            pltpu.VMEM((_NBUF, _B, _DH), jnp.float32),
            pltpu.VMEM_SHARED((_NP, _DH), jnp.float32),
            pltpu.SemaphoreType.DMA((_NBUF,)),
        ],
    )
    return f(src_t2, dst_t, h_flat)


def _inv_deg_tc(deg):
    blk = 2048

    def body(d_ref, o_ref):
        o_ref[...] = 1.0 / jnp.maximum(d_ref[...], 1.0)

    return pl.pallas_call(
        body,
        grid=(_NP // blk,),
        in_specs=[pl.BlockSpec((blk, _D), lambda i: (i, 0))],
        out_specs=pl.BlockSpec((blk, _D), lambda i: (i, 0)),
        out_shape=jax.ShapeDtypeStruct((_NP, _D), jnp.float32),
    )(deg)


def _layer_tc(h, agg, invd, w_s, w_n, bias, relu, out_w=_D):
    blk = 2000

    def body(h_ref, a_ref, i_ref, ws_ref, wn_ref, b_ref, o_ref):
        mean = a_ref[...] * i_ref[...]
        acc = jnp.dot(h_ref[...], ws_ref[...], preferred_element_type=jnp.float32)
        acc = acc + jnp.dot(mean, wn_ref[...], preferred_element_type=jnp.float32)
        acc = acc + b_ref[...]
        if relu:
            acc = jnp.maximum(acc, 0.0)
        o_ref[...] = acc[:, :out_w]

    return pl.pallas_call(
        body,
        grid=(_N // blk,),
        in_specs=[
            pl.BlockSpec((blk, _D), lambda i: (i, 0)),
            pl.BlockSpec((blk, _D), lambda i: (i, 0)),
            pl.BlockSpec((blk, _D), lambda i: (i, 0)),
            pl.BlockSpec((_D, _D), lambda i: (0, 0)),
            pl.BlockSpec((_D, _D), lambda i: (0, 0)),
            pl.BlockSpec((1, _D), lambda i: (0, 0)),
        ],
        out_specs=pl.BlockSpec((blk, out_w), lambda i: (i, 0)),
        out_shape=jax.ShapeDtypeStruct((_N, out_w), jnp.float32),
    )(h, agg, invd, w_s, w_n, bias)


def kernel(x, edge_index, W_self, W_neigh, b, W_self_out, W_neigh_out, b_out):
    src = edge_index[0]
    dst = edge_index[1]
    pad_n = _EPAD - _E
    ar = jnp.arange(pad_n, dtype=jnp.int32)
    pad_src = (ar * 131) % _N
    pad_dst = _N + (ar % _PADROWS)
    src_t = jnp.concatenate([src, pad_src]).reshape(_NS, _CH, _B)
    dst_t = jnp.concatenate([dst, pad_dst]).reshape(_NS, _CH, _B)
    src_t2 = jnp.concatenate([2 * src_t, 2 * src_t + 1], axis=0)

    def seg_sum(h):
        return _seg_sum_sc(src_t2, dst_t, h.reshape(2 * _N, _DH))

    loc = ((jnp.arange(_NS, dtype=jnp.int32) * 1280)[:, None]
           + (jnp.arange(_EPT, dtype=jnp.int32) % 1280)[None, :]).reshape(
               _NS, _CH, _B)
    src_deg = jnp.concatenate([2 * loc, 2 * loc + 1], axis=0)
    deg = _seg_sum_sc(src_deg, dst_t, jnp.ones((2 * _N, _DH), jnp.float32))
    invd = _inv_deg_tc(deg)

    w_s_out = jnp.zeros((_D, _D), jnp.float32).at[:, : W_self_out.shape[1]].set(W_self_out)
    w_n_out = jnp.zeros((_D, _D), jnp.float32).at[:, : W_neigh_out.shape[1]].set(W_neigh_out)
    b_o = jnp.zeros((1, _D), jnp.float32).at[0, : b_out.shape[0]].set(b_out)

    h = x
    n_hidden = W_self.shape[0]
    for i in range(n_hidden):
        agg = seg_sum(h)
        h = _layer_tc(h, agg, invd, W_self[i], W_neigh[i], b[i].reshape(1, _D),
                      relu=(i >= 1))
    agg = seg_sum(h)
    return _layer_tc(h, agg, invd, w_s_out, w_n_out, b_o, relu=False,
                     out_w=b_out.shape[0])
